# baseline (speedup 1.0000x reference)
import functools

import jax
import jax.numpy as jnp
from jax import lax
from jax.experimental import pallas as pl
from jax.experimental.pallas import tpu as pltpu
from jax.experimental.pallas import tpu_sc as plsc

_NN = 50000
_D = 256
_NG = 512
_NC = 2
_NS = 16
_NW = _NC * _NS
_PT = 1568
_CH = 112
_GPC = _CH // 16
_NCH_FULL = _PT // _CH
_LAST_ROWS = _NN - (_NW - 1) * _PT
_NCH_LAST = _LAST_ROWS // _CH
_TAIL = _LAST_ROWS - _NCH_LAST * _CH
_NL = _D // 16
_SW = 3 * _D + 16
_NEG = float(jnp.finfo(jnp.float32).min)


def _make_sc_pool():
    mesh = plsc.VectorSubcoreMesh(
        core_axis_name="c", subcore_axis_name="s",
        num_cores=_NC, num_subcores=_NS)

    @functools.partial(
        pl.kernel,
        out_type=jax.ShapeDtypeStruct((_NG + _NW, _SW), jnp.float32),
        mesh=mesh,
        scratch_types=[
            pltpu.VMEM((_CH, _D), jnp.float32),
            pltpu.VMEM((_CH, _D), jnp.float32),
            pltpu.VMEM((_CH + 16,), jnp.int32),
            pltpu.VMEM((_CH + 16,), jnp.int32),
            pltpu.VMEM((16,), jnp.int32),
            pltpu.VMEM((_SW,), jnp.float32),
            pltpu.VMEM((_SW,), jnp.float32),
            pltpu.SemaphoreType.DMA,
            pltpu.SemaphoreType.DMA,
            pltpu.SemaphoreType.DMA,
            pltpu.SemaphoreType.DMA,
        ],
    )
    def sc_pool(x_hbm, ids_hbm, stats_hbm, xbuf0, xbuf1, idbuf0, idbuf1,
                pbuf, stage, zrow, sx0, si0, sx1, si1):
        w = lax.axis_index("s") * _NC + lax.axis_index("c")
        start = w * _PT

        poff = pl.multiple_of(jnp.maximum(start - 8, 0), 8)
        pltpu.sync_copy(ids_hbm.at[pl.ds(poff, 16)], pbuf)
        pv = pbuf[pl.ds(0, 16)]
        pend0 = jnp.where(w == 0, jnp.int32(0),
                          (pv[7] == pv[8]).astype(jnp.int32))

        zero16 = jnp.zeros((16,), jnp.float32)
        neg16 = jnp.full((16,), _NEG, jnp.float32)
        sent16 = jnp.full((16,), -1, jnp.int32)
        idbuf0[pl.ds(_CH, 16)] = sent16
        idbuf1[pl.ds(_CH, 16)] = sent16
        for q in range(_SW // 16):
            zrow[pl.ds(q * 16, 16)] = zero16

        def zero_fill(lo, hi):
            def zbody(s, c):
                pltpu.sync_copy(zrow, stats_hbm.at[s])
                return c
            lax.fori_loop(lo, hi, zbody, jnp.int32(0))

        gap_lo = jnp.where(w == 0, jnp.int32(0), pv[7] + 1)
        gap_hi = jnp.where(w == 0, pv[0], pv[8])
        zero_fill(gap_lo, gap_hi)

        carry = (
            jnp.int32(-1),
            pend0,
            jnp.float32(0.0),
            [zero16] * _NL, [zero16] * _NL, [neg16] * _NL,
        )

        def flush_stores(cur_id, pending, cnt, sums, sqs, mxs, gap_hi):
            for j in range(_NL):
                stage[pl.ds(j * 16, 16)] = sums[j]
                stage[pl.ds(_D + j * 16, 16)] = sqs[j]
                stage[pl.ds(2 * _D + j * 16, 16)] = mxs[j]
            stage[pl.ds(3 * _D, 16)] = jnp.full((16,), cnt, jnp.float32)
            dst = jnp.where(pending == 1, _NG + w, cur_id)
            pltpu.sync_copy(stage, stats_hbm.at[dst])
            zero_fill(cur_id + 1, gap_hi)

        def make_acc_row(xb):
            def acc_row(r, sums, sqs, mxs):
                ns, nq, nm = [], [], []
                for j in range(_NL):
                    xv = xb[r, pl.ds(j * 16, 16)]
                    ns.append(sums[j] + xv)
                    nq.append(sqs[j] + xv * xv)
                    nm.append(jnp.maximum(mxs[j], xv))
                return ns, nq, nm
            return acc_row

        def make_acc_row4(xb):
            def acc_row4(r, sums, sqs, mxs):
                ns, nq, nm = [], [], []
                for j in range(_NL):
                    x0 = xb[r, pl.ds(j * 16, 16)]
                    x1 = xb[r + 1, pl.ds(j * 16, 16)]
                    x2 = xb[r + 2, pl.ds(j * 16, 16)]
                    x3 = xb[r + 3, pl.ds(j * 16, 16)]
                    ns.append(sums[j] + ((x0 + x1) + (x2 + x3)))
                    nq.append(sqs[j] + ((x0 * x0 + x1 * x1) +
                                        (x2 * x2 + x3 * x3)))
                    nm.append(jnp.maximum(
                        jnp.maximum(jnp.maximum(mxs[j], x0), x1),
                        jnp.maximum(x2, x3)))
                return ns, nq, nm
            return acc_row4

        def make_window_body(xb, idb):
          acc_row = make_acc_row(xb)
          acc_row4 = make_acc_row4(xb)
          idbuf = idb

          def window_body(g, carry):
            cur_id, pending, cnt, sums, sqs, mxs = carry
            q0 = g * 16
            widw = idbuf[pl.ds(q0, 16)]
            lanes = [widw[i] for i in range(16)]
            last_id = lanes[15]
            pre_n = jnp.int32(0)
            suf_n = jnp.int32(0)
            for i in range(16):
                pre_n = pre_n + (lanes[i] == cur_id).astype(jnp.int32)
                suf_n = suf_n + (lanes[i] == last_id).astype(jnp.int32)

            def pbody4(k, acc):
                return acc_row4(q0 + 4 * k, *acc)

            def pbody(k, acc):
                return acc_row(q0 + k, *acc)

            n4 = lax.shift_right_logical(pre_n, 1 + 1)
            sums, sqs, mxs = lax.fori_loop(0, n4, pbody4,
                                           (sums, sqs, mxs))
            sums, sqs, mxs = lax.fori_loop(4 * n4, pre_n, pbody,
                                           (sums, sqs, mxs))
            cnt = cnt + pre_n.astype(jnp.float32)

            nxt = lanes[15]
            for i in range(14, -1, -1):
                nxt = jnp.where(lanes[i] != cur_id, lanes[i], nxt)
            do_flush = jnp.logical_and(pre_n < 16, cnt > 0.5)

            @pl.when(do_flush)
            def _():
                flush_stores(cur_id, pending, cnt, sums, sqs, mxs, nxt)

            pending = jnp.where(do_flush, jnp.int32(0), pending)
            cnt = jnp.where(do_flush, 0.0, cnt)

            def ibody(k, icarry):
                cur_id, pending, cnt, sums, sqs, mxs = icarry
                rid = lanes[15]
                for i in range(14, -1, -1):
                    rid = jnp.where(k == i, lanes[i], rid)
                is_new = rid != cur_id
                do_f = jnp.logical_and(is_new, cnt > 0.5)

                @pl.when(do_f)
                def _():
                    flush_stores(cur_id, pending, cnt, sums, sqs, mxs, rid)

                keep = jnp.where(is_new, 0.0, 1.0)
                pending = jnp.where(do_f, jnp.int32(0), pending)
                sums = [s * keep for s in sums]
                sqs = [s * keep for s in sqs]
                mxs = [jnp.where(is_new, neg16, m) for m in mxs]
                sums, sqs, mxs = acc_row(q0 + k, sums, sqs, mxs)
                return (rid, pending, cnt * keep + 1.0, sums, sqs, mxs)

            ilo = jnp.maximum(pre_n, 16 - suf_n)
            cur_id, pending, cnt, sums, sqs, mxs = lax.fori_loop(
                pre_n, 16 - suf_n, ibody,
                (cur_id, pending, cnt, sums, sqs, mxs))

            is_new = last_id != cur_id
            do_f2 = jnp.logical_and(is_new, cnt > 0.5)

            @pl.when(do_f2)
            def _(cur_id=cur_id, pending=pending, cnt=cnt,
                  sums=sums, sqs=sqs, mxs=mxs):
                flush_stores(cur_id, pending, cnt, sums, sqs, mxs, last_id)

            pending = jnp.where(do_f2, jnp.int32(0), pending)
            keep = jnp.where(is_new, 0.0, 1.0)
            sums = [s * keep for s in sums]
            sqs = [s * keep for s in sqs]
            mxs = [jnp.where(is_new, neg16, m) for m in mxs]

            def sbody(k, acc):
                return acc_row(q0 + k, *acc)
            sums, sqs, mxs = lax.fori_loop(ilo, 16, sbody,
                                           (sums, sqs, mxs))
            cnt = cnt * keep + (16 - ilo).astype(jnp.float32)
            return (last_id, pending, cnt, sums, sqs, mxs)

          return window_body

        wb0 = make_window_body(xbuf0, idbuf0)
        wb1 = make_window_body(xbuf1, idbuf1)

        def dma_row0(c):
            return pl.multiple_of(
                jnp.minimum(start + c * _CH, _NN - _CH), 16)

        def start_dma(c, xb, idb, sx, si):
            row0 = dma_row0(c)
            pltpu.async_copy(x_hbm.at[pl.ds(row0, _CH)], xb, sx)
            pltpu.async_copy(ids_hbm.at[pl.ds(row0, _CH)],
                             idb.at[pl.ds(0, _CH)], si)

        def wait_dma(c, xb, idb, sx, si):
            row0 = dma_row0(c)
            pltpu.make_async_copy(x_hbm.at[pl.ds(row0, _CH)], xb, sx).wait()
            pltpu.make_async_copy(ids_hbm.at[pl.ds(row0, _CH)],
                                  idb.at[pl.ds(0, _CH)], si).wait()

        nch = jnp.where(w == _NW - 1, _NCH_LAST, _NCH_FULL)
        start_dma(jnp.int32(0), xbuf0, idbuf0, sx0, si0)

        def pair_body(p, carry):
            c0 = 2 * p
            wait_dma(c0, xbuf0, idbuf0, sx0, si0)
            start_dma(c0 + 1, xbuf1, idbuf1, sx1, si1)
            carry = lax.fori_loop(0, _GPC, wb0, carry)
            wait_dma(c0 + 1, xbuf1, idbuf1, sx1, si1)
            start_dma(c0 + 2, xbuf0, idbuf0, sx0, si0)
            return lax.fori_loop(0, _GPC, wb1, carry)

        carry = lax.fori_loop(0, nch // 2, pair_body, carry)
        wait_dma(nch, xbuf0, idbuf0, sx0, si0)

        trow0 = (_NW - 1) * _PT + _NCH_LAST * _CH
        pltpu.sync_copy(x_hbm.at[pl.ds(trow0, _TAIL)],
                        xbuf0.at[pl.ds(0, _TAIL)])
        pltpu.sync_copy(ids_hbm.at[pl.ds(trow0, _TAIL)],
                        idbuf0.at[pl.ds(0, _TAIL)])
        for t in range(_TAIL // 16, _CH // 16):
            idbuf0[pl.ds(t * 16, 16)] = sent16
        ntail = jnp.where(w == _NW - 1, _TAIL // 16, 0)
        carry = lax.fori_loop(0, ntail, wb0, carry)

        cur_id, pending, cnt, sums, sqs, mxs = carry
        final_hi = jnp.where(w == _NW - 1, jnp.int32(_NG), cur_id + 1)
        flush_stores(cur_id, pending, cnt, sums, sqs, mxs, final_hi)

        @pl.when(pend0 == 0)
        def _():
            for j in range(_NL):
                stage[pl.ds(j * 16, 16)] = zero16
                stage[pl.ds(_D + j * 16, 16)] = zero16
                stage[pl.ds(2 * _D + j * 16, 16)] = neg16
            stage[pl.ds(3 * _D, 16)] = zero16
            pltpu.sync_copy(stage, stats_hbm.at[_NG + w])

    return sc_pool


def _epilogue_kernel(stats_ref, fid_row_ref, fid_col_ref,
                     w_ref, b_ref, g_ref, be_ref, o_ref):
    stats = stats_ref[...]
    main = stats[:_NG]
    side = stats[_NG:]

    fid_row = fid_row_ref[0]
    fid_col = fid_col_ref[0]
    segs = jax.lax.broadcasted_iota(jnp.int32, (_NG, 1), 0)
    onehot = (segs == fid_row).astype(jnp.float32)
    side_add = jax.lax.dot_general(
        onehot, side, (((1,), (0,)), ((), ())),
        precision=jax.lax.Precision.HIGHEST,
        preferred_element_type=jnp.float32)

    s = main[:, :_D] + side_add[:, :_D]
    sq = main[:, _D:2 * _D] + side_add[:, _D:2 * _D]
    cnt = main[:, 3 * _D:3 * _D + 1] + side_add[:, 3 * _D:3 * _D + 1]
    present = cnt > 0.0

    m = side[:, 2 * _D:3 * _D]
    k = 1
    while k < _NW:
        pm = jnp.concatenate(
            [jnp.full((k, _D), _NEG, jnp.float32), m[:-k]], axis=0)
        pid = jnp.concatenate(
            [jnp.full((k, 1), -1, jnp.int32), fid_col[:-k]], axis=0)
        m = jnp.where(pid == fid_col, jnp.maximum(m, pm), m)
        k *= 2
    nid = jnp.concatenate(
        [fid_row[:, 1:], jnp.full((1, 1), -2, jnp.int32)], axis=1)
    last = (nid != fid_row).astype(jnp.float32)
    sel = onehot * last
    mside = jax.lax.dot_general(sel, m, (((1,), (0,)), ((), ())),
                                precision=jax.lax.Precision.HIGHEST,
                                preferred_element_type=jnp.float32)
    side_present = jnp.sum(sel, axis=1, keepdims=True) > 0.0
    xmax = jnp.where(
        present,
        jnp.maximum(main[:, 2 * _D:3 * _D],
                    jnp.where(side_present, mside, _NEG)),
        _NEG)

    mean = s / jnp.maximum(cnt, 1.0)
    var_sum = jnp.maximum(sq - s * mean, 0.0)
    denom = jnp.maximum(cnt - 1.0, 1.0)
    std = jnp.sqrt(var_sum / denom)

    pooled = jnp.concatenate([mean, xmax, std], axis=1)
    h = jax.lax.dot_general(pooled, w_ref[...], (((1,), (0,)), ((), ())),
                            preferred_element_type=jnp.float32)
    h = h + b_ref[...]
    mu = jnp.mean(h, axis=1, keepdims=True)
    var = jnp.mean((h - mu) ** 2, axis=1, keepdims=True)
    hn = (h - mu) * jax.lax.rsqrt(var + 1e-5) * g_ref[...] + be_ref[...]
    o_ref[...] = jnp.where(hn >= 0, hn, 0.01 * hn)


def kernel(x, batch, W, b, gamma, beta):
    ids = batch.astype(jnp.int32)
    starts = jnp.arange(_NW, dtype=jnp.int32) * _PT
    fid = ids[starts]

    stats = _make_sc_pool()(x, ids)

    out = pl.pallas_call(
        _epilogue_kernel,
        in_specs=[
            pl.BlockSpec((_NG + _NW, _SW), lambda: (0, 0)),
            pl.BlockSpec((1, 1, _NW), lambda: (0, 0, 0)),
            pl.BlockSpec((1, _NW, 1), lambda: (0, 0, 0)),
            pl.BlockSpec((3 * _D, _D), lambda: (0, 0)),
            pl.BlockSpec((1, _D), lambda: (0, 0)),
            pl.BlockSpec((1, _D), lambda: (0, 0)),
            pl.BlockSpec((1, _D), lambda: (0, 0)),
        ],
        out_specs=pl.BlockSpec((_NG, _D), lambda: (0, 0)),
        out_shape=jax.ShapeDtypeStruct((_NG, _D), jnp.float32),
    )(stats, fid.reshape(1, 1, _NW), fid.reshape(1, _NW, 1),
      W, b.reshape(1, _D), gamma.reshape(1, _D), beta.reshape(1, _D))
    return out


# speedup vs baseline: 1.3326x; 1.3326x over previous
import functools

import jax
import jax.numpy as jnp
from jax import lax
from jax.experimental import pallas as pl
from jax.experimental.pallas import tpu as pltpu
from jax.experimental.pallas import tpu_sc as plsc

_NN = 50000
_D = 256
_NG = 512
_NC = 2
_NS = 16
_NW = _NC * _NS
_PT = 1568
_CH = 112
_GPC = _CH // 16
_NCH_FULL = _PT // _CH
_LAST_ROWS = _NN - (_NW - 1) * _PT
_NCH_LAST = _LAST_ROWS // _CH
_TAIL = _LAST_ROWS - _NCH_LAST * _CH
_NL = _D // 16
_SW = 3 * _D + 16
_NEG = float(jnp.finfo(jnp.float32).min)


def _make_sc_pool():
    mesh = plsc.VectorSubcoreMesh(
        core_axis_name="c", subcore_axis_name="s",
        num_cores=_NC, num_subcores=_NS)

    @functools.partial(
        pl.kernel,
        out_type=jax.ShapeDtypeStruct((_NG + _NW, _SW), jnp.float32),
        mesh=mesh,
        scratch_types=[
            pltpu.VMEM((_CH, _D), jnp.float32),
            pltpu.VMEM((_CH, _D), jnp.float32),
            pltpu.VMEM((_CH + 16,), jnp.int32),
            pltpu.VMEM((_CH + 16,), jnp.int32),
            pltpu.VMEM((16,), jnp.int32),
            pltpu.VMEM((_SW,), jnp.float32),
            pltpu.VMEM((_SW,), jnp.float32),
            pltpu.SemaphoreType.DMA,
            pltpu.SemaphoreType.DMA,
            pltpu.SemaphoreType.DMA,
            pltpu.SemaphoreType.DMA,
        ],
    )
    def sc_pool(x_hbm, ids_hbm, stats_hbm, xbuf0, xbuf1, idbuf0, idbuf1,
                pbuf, stage, zrow, sx0, si0, sx1, si1):
        w = lax.axis_index("s") * _NC + lax.axis_index("c")
        start = w * _PT

        poff = pl.multiple_of(jnp.maximum(start - 8, 0), 8)
        pltpu.sync_copy(ids_hbm.at[pl.ds(poff, 16)], pbuf)
        pv = pbuf[pl.ds(0, 16)]
        pend0 = jnp.where(w == 0, jnp.int32(0),
                          (pv[7] == pv[8]).astype(jnp.int32))

        zero16 = jnp.zeros((16,), jnp.float32)
        neg16 = jnp.full((16,), _NEG, jnp.float32)
        sent16 = jnp.full((16,), -1, jnp.int32)
        idbuf0[pl.ds(_CH, 16)] = sent16
        idbuf1[pl.ds(_CH, 16)] = sent16
        for q in range(_SW // 16):
            zrow[pl.ds(q * 16, 16)] = zero16

        def zero_fill(lo, hi):
            def zbody(s, c):
                pltpu.sync_copy(zrow, stats_hbm.at[s])
                return c
            lax.fori_loop(lo, hi, zbody, jnp.int32(0))

        gap_lo = jnp.where(w == 0, jnp.int32(0), pv[7] + 1)
        gap_hi = jnp.where(w == 0, pv[0], pv[8])
        zero_fill(gap_lo, gap_hi)

        carry = (
            jnp.int32(-1),
            pend0,
            jnp.float32(0.0),
            [zero16] * _NL, [zero16] * _NL, [neg16] * _NL,
        )

        def flush_stores(cur_id, pending, cnt, sums, sqs, mxs, gap_hi):
            for j in range(_NL):
                stage[pl.ds(j * 16, 16)] = sums[j]
                stage[pl.ds(_D + j * 16, 16)] = sqs[j]
                stage[pl.ds(2 * _D + j * 16, 16)] = mxs[j]
            stage[pl.ds(3 * _D, 16)] = jnp.full((16,), cnt, jnp.float32)
            dst = jnp.where(pending == 1, _NG + w, cur_id)
            pltpu.sync_copy(stage, stats_hbm.at[dst])
            zero_fill(cur_id + 1, gap_hi)

        def make_acc_row(xb):
            def acc_row(r, sums, sqs, mxs):
                ns, nq, nm = [], [], []
                for j in range(_NL):
                    xv = xb[r, pl.ds(j * 16, 16)]
                    ns.append(sums[j] + xv)
                    nq.append(sqs[j] + xv * xv)
                    nm.append(jnp.maximum(mxs[j], xv))
                return ns, nq, nm
            return acc_row

        def make_window_body(xb, idb):
          acc_row = make_acc_row(xb)
          idbuf = idb

          def window_body(g, carry):
            cur_id, pending, cnt, sums, sqs, mxs = carry
            q0 = g * 16
            widw = idbuf[pl.ds(q0, 16)]
            lanes = [widw[i] for i in range(16)]
            last_id = lanes[15]
            pre_n = jnp.int32(0)
            suf_n = jnp.int32(0)
            for i in range(16):
                pre_n = pre_n + (lanes[i] == cur_id).astype(jnp.int32)
                suf_n = suf_n + (lanes[i] == last_id).astype(jnp.int32)

            def pbody(k, acc):
                return acc_row(q0 + k, *acc)
            sums, sqs, mxs = lax.fori_loop(0, pre_n, pbody,
                                           (sums, sqs, mxs))
            cnt = cnt + pre_n.astype(jnp.float32)

            nxt = lanes[15]
            for i in range(14, -1, -1):
                nxt = jnp.where(lanes[i] != cur_id, lanes[i], nxt)
            do_flush = jnp.logical_and(pre_n < 16, cnt > 0.5)

            @pl.when(do_flush)
            def _():
                flush_stores(cur_id, pending, cnt, sums, sqs, mxs, nxt)

            pending = jnp.where(do_flush, jnp.int32(0), pending)
            cnt = jnp.where(do_flush, 0.0, cnt)

            def ibody(k, icarry):
                cur_id, pending, cnt, sums, sqs, mxs = icarry
                rid = lanes[15]
                for i in range(14, -1, -1):
                    rid = jnp.where(k == i, lanes[i], rid)
                is_new = rid != cur_id
                do_f = jnp.logical_and(is_new, cnt > 0.5)

                @pl.when(do_f)
                def _():
                    flush_stores(cur_id, pending, cnt, sums, sqs, mxs, rid)

                keep = jnp.where(is_new, 0.0, 1.0)
                pending = jnp.where(do_f, jnp.int32(0), pending)
                sums = [s * keep for s in sums]
                sqs = [s * keep for s in sqs]
                mxs = [jnp.where(is_new, neg16, m) for m in mxs]
                sums, sqs, mxs = acc_row(q0 + k, sums, sqs, mxs)
                return (rid, pending, cnt * keep + 1.0, sums, sqs, mxs)

            ilo = jnp.maximum(pre_n, 16 - suf_n)
            cur_id, pending, cnt, sums, sqs, mxs = lax.fori_loop(
                pre_n, 16 - suf_n, ibody,
                (cur_id, pending, cnt, sums, sqs, mxs))

            is_new = last_id != cur_id
            do_f2 = jnp.logical_and(is_new, cnt > 0.5)

            @pl.when(do_f2)
            def _(cur_id=cur_id, pending=pending, cnt=cnt,
                  sums=sums, sqs=sqs, mxs=mxs):
                flush_stores(cur_id, pending, cnt, sums, sqs, mxs, last_id)

            pending = jnp.where(do_f2, jnp.int32(0), pending)
            keep = jnp.where(is_new, 0.0, 1.0)
            sums = [s * keep for s in sums]
            sqs = [s * keep for s in sqs]
            mxs = [jnp.where(is_new, neg16, m) for m in mxs]

            def sbody(k, acc):
                return acc_row(q0 + k, *acc)
            sums, sqs, mxs = lax.fori_loop(ilo, 16, sbody,
                                           (sums, sqs, mxs))
            cnt = cnt * keep + (16 - ilo).astype(jnp.float32)
            return (last_id, pending, cnt, sums, sqs, mxs)

          return window_body

        wb0 = make_window_body(xbuf0, idbuf0)
        wb1 = make_window_body(xbuf1, idbuf1)

        def dma_row0(c):
            return pl.multiple_of(
                jnp.minimum(start + c * _CH, _NN - _CH), 16)

        def start_dma(c, xb, idb, sx, si):
            row0 = dma_row0(c)
            pltpu.async_copy(x_hbm.at[pl.ds(row0, _CH)], xb, sx)
            pltpu.async_copy(ids_hbm.at[pl.ds(row0, _CH)],
                             idb.at[pl.ds(0, _CH)], si)

        def wait_dma(c, xb, idb, sx, si):
            row0 = dma_row0(c)
            pltpu.make_async_copy(x_hbm.at[pl.ds(row0, _CH)], xb, sx).wait()
            pltpu.make_async_copy(ids_hbm.at[pl.ds(row0, _CH)],
                                  idb.at[pl.ds(0, _CH)], si).wait()

        nch = jnp.where(w == _NW - 1, _NCH_LAST, _NCH_FULL)
        start_dma(jnp.int32(0), xbuf0, idbuf0, sx0, si0)

        def pair_body(p, carry):
            c0 = 2 * p
            wait_dma(c0, xbuf0, idbuf0, sx0, si0)
            start_dma(c0 + 1, xbuf1, idbuf1, sx1, si1)
            carry = lax.fori_loop(0, _GPC, wb0, carry)
            wait_dma(c0 + 1, xbuf1, idbuf1, sx1, si1)
            start_dma(c0 + 2, xbuf0, idbuf0, sx0, si0)
            return lax.fori_loop(0, _GPC, wb1, carry)

        carry = lax.fori_loop(0, nch // 2, pair_body, carry)
        wait_dma(nch, xbuf0, idbuf0, sx0, si0)

        trow0 = (_NW - 1) * _PT + _NCH_LAST * _CH
        pltpu.sync_copy(x_hbm.at[pl.ds(trow0, _TAIL)],
                        xbuf0.at[pl.ds(0, _TAIL)])
        pltpu.sync_copy(ids_hbm.at[pl.ds(trow0, _TAIL)],
                        idbuf0.at[pl.ds(0, _TAIL)])
        for t in range(_TAIL // 16, _CH // 16):
            idbuf0[pl.ds(t * 16, 16)] = sent16
        ntail = jnp.where(w == _NW - 1, _TAIL // 16, 0)
        carry = lax.fori_loop(0, ntail, wb0, carry)

        cur_id, pending, cnt, sums, sqs, mxs = carry
        final_hi = jnp.where(w == _NW - 1, jnp.int32(_NG), cur_id + 1)
        flush_stores(cur_id, pending, cnt, sums, sqs, mxs, final_hi)

        @pl.when(pend0 == 0)
        def _():
            for j in range(_NL):
                stage[pl.ds(j * 16, 16)] = zero16
                stage[pl.ds(_D + j * 16, 16)] = zero16
                stage[pl.ds(2 * _D + j * 16, 16)] = neg16
            stage[pl.ds(3 * _D, 16)] = zero16
            pltpu.sync_copy(stage, stats_hbm.at[_NG + w])

    return sc_pool


def _epilogue_kernel(stats_ref, fid_row_ref, fid_col_ref,
                     w_ref, b_ref, g_ref, be_ref, o_ref):
    stats = stats_ref[...]
    main = stats[:_NG]
    side = stats[_NG:]

    fid_row = fid_row_ref[0]
    fid_col = fid_col_ref[0]
    segs = jax.lax.broadcasted_iota(jnp.int32, (_NG, 1), 0)
    onehot = (segs == fid_row).astype(jnp.float32)
    side_add = jax.lax.dot_general(
        onehot, side, (((1,), (0,)), ((), ())),
        precision=jax.lax.Precision.HIGHEST,
        preferred_element_type=jnp.float32)

    s = main[:, :_D] + side_add[:, :_D]
    sq = main[:, _D:2 * _D] + side_add[:, _D:2 * _D]
    cnt = main[:, 3 * _D:3 * _D + 1] + side_add[:, 3 * _D:3 * _D + 1]
    present = cnt > 0.0

    m = side[:, 2 * _D:3 * _D]
    k = 1
    while k < _NW:
        pm = jnp.concatenate(
            [jnp.full((k, _D), _NEG, jnp.float32), m[:-k]], axis=0)
        pid = jnp.concatenate(
            [jnp.full((k, 1), -1, jnp.int32), fid_col[:-k]], axis=0)
        m = jnp.where(pid == fid_col, jnp.maximum(m, pm), m)
        k *= 2
    nid = jnp.concatenate(
        [fid_row[:, 1:], jnp.full((1, 1), -2, jnp.int32)], axis=1)
    last = (nid != fid_row).astype(jnp.float32)
    sel = onehot * last
    mside = jax.lax.dot_general(sel, m, (((1,), (0,)), ((), ())),
                                precision=jax.lax.Precision.HIGHEST,
                                preferred_element_type=jnp.float32)
    side_present = jnp.sum(sel, axis=1, keepdims=True) > 0.0
    xmax = jnp.where(
        present,
        jnp.maximum(main[:, 2 * _D:3 * _D],
                    jnp.where(side_present, mside, _NEG)),
        _NEG)

    mean = s / jnp.maximum(cnt, 1.0)
    var_sum = jnp.maximum(sq - s * mean, 0.0)
    denom = jnp.maximum(cnt - 1.0, 1.0)
    std = jnp.sqrt(var_sum / denom)

    pooled = jnp.concatenate([mean, xmax, std], axis=1)
    h = jax.lax.dot_general(pooled, w_ref[...], (((1,), (0,)), ((), ())),
                            preferred_element_type=jnp.float32)
    h = h + b_ref[...]
    mu = jnp.mean(h, axis=1, keepdims=True)
    var = jnp.mean((h - mu) ** 2, axis=1, keepdims=True)
    hn = (h - mu) * jax.lax.rsqrt(var + 1e-5) * g_ref[...] + be_ref[...]
    o_ref[...] = jnp.where(hn >= 0, hn, 0.01 * hn)


def kernel(x, batch, W, b, gamma, beta):
    ids = batch.astype(jnp.int32)
    starts = jnp.arange(_NW, dtype=jnp.int32) * _PT
    fid = ids[starts]

    stats = _make_sc_pool()(x, ids)

    out = pl.pallas_call(
        _epilogue_kernel,
        in_specs=[
            pl.BlockSpec((_NG + _NW, _SW), lambda: (0, 0)),
            pl.BlockSpec((1, 1, _NW), lambda: (0, 0, 0)),
            pl.BlockSpec((1, _NW, 1), lambda: (0, 0, 0)),
            pl.BlockSpec((3 * _D, _D), lambda: (0, 0)),
            pl.BlockSpec((1, _D), lambda: (0, 0)),
            pl.BlockSpec((1, _D), lambda: (0, 0)),
            pl.BlockSpec((1, _D), lambda: (0, 0)),
        ],
        out_specs=pl.BlockSpec((_NG, _D), lambda: (0, 0)),
        out_shape=jax.ShapeDtypeStruct((_NG, _D), jnp.float32),
    )(stats, fid.reshape(1, 1, _NW), fid.reshape(1, _NW, 1),
      W, b.reshape(1, _D), gamma.reshape(1, _D), beta.reshape(1, _D))
    return out
